# Initial kernel scaffold; baseline (speedup 1.0000x reference)
import jax
import jax.numpy as jnp
from jax.experimental import pallas as pl


def kernel(input, W_in, W_h, W_out):
    raise NotImplementedError("write your pallas kernel here")


# speedup vs baseline: 29.9159x; 29.9159x over previous
import functools

import jax
import jax.numpy as jnp
from jax import lax
from jax.experimental import pallas as pl
from jax.experimental.pallas import tpu as pltpu
from jax.experimental.pallas import tpu_sc as plsc

B, S, D_IN, HID, VOCAB, MAX_LEN = 16, 8, 256, 512, 128, 512
R = B * S
FLAT = S * MAX_LEN


def _rnn_body(xt_ref, wint_ref, wht_ref, woutt_ref,
              tok_ref, lp_ref, ent_ref, dest_ref):
    h0 = jnp.tanh(jnp.dot(wint_ref[...], xt_ref[...],
                          preferred_element_type=jnp.float32))
    wht = wht_ref[...]
    woutt = woutt_ref[...]
    vocab_iota = lax.broadcasted_iota(jnp.int32, (VOCAB, R), 0)

    def step(t, h):
        h = jnp.tanh(jnp.dot(wht, h, preferred_element_type=jnp.float32))
        logits = jnp.dot(woutt, h, preferred_element_type=jnp.float32)
        m = jnp.max(logits, axis=0, keepdims=True)
        shifted = logits - m
        e = jnp.exp(shifted)
        ssum = jnp.sum(e, axis=0, keepdims=True)
        tok = jnp.min(jnp.where(logits == m, vocab_iota, VOCAB),
                      axis=0, keepdims=True)
        logs = jnp.log(ssum)
        lp = -logs
        ent = logs - jnp.sum(e * shifted, axis=0, keepdims=True) / ssum
        tok_ref[pl.ds(t, 1), :] = tok
        lp_ref[pl.ds(t, 1), :] = lp
        ent_ref[pl.ds(t, 1), :] = ent
        return h

    lax.fori_loop(0, MAX_LEN, step, h0)

    toks = tok_ref[...]
    pos = lax.broadcasted_iota(jnp.int32, (MAX_LEN, R), 0)
    fz = jnp.min(jnp.where(toks == 0, pos, MAX_LEN), axis=0, keepdims=True)
    length = jnp.minimum(fz + 1, MAX_LEN)
    p = length - 1
    p_f = p.astype(jnp.float32)

    r0 = lax.broadcasted_iota(jnp.int32, (R, R), 0)
    r1 = lax.broadcasted_iota(jnp.int32, (R, R), 1)
    same_b = (r0 // S) == (r1 // S)
    mt_strict = jnp.where(same_b & (r0 < r1), 1.0, 0.0).astype(jnp.float32)
    mt_block = jnp.where(same_b, 1.0, 0.0).astype(jnp.float32)
    fo = jnp.dot(p_f, mt_strict, preferred_element_type=jnp.float32,
                 precision=lax.Precision.HIGHEST)
    tf = jnp.dot(p_f, mt_block, preferred_element_type=jnp.float32,
                 precision=lax.Precision.HIGHEST)
    s_idx = lax.broadcasted_iota(jnp.int32, (1, R), 1) % S
    lo = s_idx.astype(jnp.float32) * MAX_LEN - fo
    pos_f = pos.astype(jnp.float32)
    dest_f = jnp.where(pos < p, fo + pos_f, tf + lo + pos_f - p_f)
    dest_ref[...] = dest_f.astype(jnp.int32)


_rnn_call = pl.pallas_call(
    _rnn_body,
    out_shape=[
        jax.ShapeDtypeStruct((MAX_LEN, R), jnp.int32),
        jax.ShapeDtypeStruct((MAX_LEN, R), jnp.float32),
        jax.ShapeDtypeStruct((MAX_LEN, R), jnp.float32),
        jax.ShapeDtypeStruct((MAX_LEN, R), jnp.int32),
    ],
)


_NC, _NS = 2, 16
_NW = _NC * _NS


def _sc_scatter_body(vals_f_hbm, toks_hbm, dest_hbm, out_f_hbm, out_i_hbm,
                     val_v, tok_v, orow_f, orow_i, idx_v):
    wid = lax.axis_index("s") * _NC + lax.axis_index("c")
    b = lax.rem(wid, B)
    pltpu.sync_copy(vals_f_hbm.at[wid], val_v)
    pltpu.sync_copy(dest_hbm.at[b], idx_v)

    def body_f(i, carry):
        sl = pl.ds(i * 16, 16)
        plsc.store_scatter(orow_f, [idx_v[sl]], val_v[sl])
        return carry

    lax.fori_loop(0, FLAT // 16, body_f, 0)
    pltpu.sync_copy(orow_f, out_f_hbm.at[wid])

    @pl.when(wid < B)
    def _():
        pltpu.sync_copy(toks_hbm.at[wid], tok_v)

        def body_i(i, carry):
            sl = pl.ds(i * 16, 16)
            plsc.store_scatter(orow_i, [idx_v[sl]], tok_v[sl])
            return carry

        lax.fori_loop(0, FLAT // 16, body_i, 0)
        pltpu.sync_copy(orow_i, out_i_hbm.at[wid])


@functools.cache
def _make_sc_scatter():
    return pl.kernel(
        _sc_scatter_body,
        mesh=plsc.VectorSubcoreMesh(
            core_axis_name="c", subcore_axis_name="s",
            num_cores=_NC, num_subcores=_NS),
        out_type=[
            jax.ShapeDtypeStruct((2 * B, FLAT), jnp.float32),
            jax.ShapeDtypeStruct((B, FLAT), jnp.int32),
        ],
        scratch_types=[
            pltpu.VMEM((FLAT,), jnp.float32),
            pltpu.VMEM((FLAT,), jnp.int32),
            pltpu.VMEM((FLAT,), jnp.float32),
            pltpu.VMEM((FLAT,), jnp.int32),
            pltpu.VMEM((FLAT,), jnp.int32),
        ],
        compiler_params=pltpu.CompilerParams(needs_layout_passes=False),
    )


@jax.jit
def kernel(input, W_in, W_h, W_out):
    xt = input.reshape(R, D_IN).T
    toks_t, lp_t, ent_t, dest_t = _rnn_call(xt, W_in.T, W_h.T, W_out.T)
    toks = toks_t.T.reshape(B, FLAT)
    dest = dest_t.T.reshape(B, FLAT)
    vals_f = jnp.concatenate(
        [lp_t.T.reshape(B, FLAT), ent_t.T.reshape(B, FLAT)], axis=0)
    out_f, out_i = _make_sc_scatter()(vals_f, toks, dest)
    return (out_i, out_f[:B], out_f[B:])


# speedup vs baseline: 37.4573x; 1.2521x over previous
import functools

import jax
import jax.numpy as jnp
from jax import lax
from jax.experimental import pallas as pl
from jax.experimental.pallas import tpu as pltpu
from jax.experimental.pallas import tpu_sc as plsc

B, S, D_IN, HID, VOCAB, MAX_LEN = 16, 8, 256, 512, 128, 512
R = B * S
FLAT = S * MAX_LEN


def _rnn_body(xt_ref, wint_ref, wht_ref, woutt_ref,
              tok_ref, lp_ref, ent_ref, dest_ref):
    h0 = jnp.tanh(jnp.dot(wint_ref[...], xt_ref[...],
                          preferred_element_type=jnp.float32))
    wht = wht_ref[...]
    woutt = woutt_ref[...]
    vocab_iota = lax.broadcasted_iota(jnp.int32, (VOCAB, R), 0)

    def step(t, h):
        h = jnp.tanh(jnp.dot(wht, h, preferred_element_type=jnp.float32))
        logits = jnp.dot(woutt, h, preferred_element_type=jnp.float32)
        m = jnp.max(logits, axis=0, keepdims=True)
        shifted = logits - m
        e = jnp.exp(shifted)
        ssum = jnp.sum(e, axis=0, keepdims=True)
        tok = jnp.min(jnp.where(logits == m, vocab_iota, VOCAB),
                      axis=0, keepdims=True)
        logs = jnp.log(ssum)
        lp = -logs
        ent = logs - jnp.sum(e * shifted, axis=0, keepdims=True) / ssum
        tok_ref[pl.ds(t, 1), :] = tok
        lp_ref[pl.ds(t, 1), :] = lp
        ent_ref[pl.ds(t, 1), :] = ent
        return h

    lax.fori_loop(0, MAX_LEN, step, h0, unroll=4)

    toks = tok_ref[...]
    pos = lax.broadcasted_iota(jnp.int32, (MAX_LEN, R), 0)
    fz = jnp.min(jnp.where(toks == 0, pos, MAX_LEN), axis=0, keepdims=True)
    length = jnp.minimum(fz + 1, MAX_LEN)
    p = length - 1
    p_f = p.astype(jnp.float32)

    r0 = lax.broadcasted_iota(jnp.int32, (R, R), 0)
    r1 = lax.broadcasted_iota(jnp.int32, (R, R), 1)
    same_b = (r0 // S) == (r1 // S)
    mt_strict = jnp.where(same_b & (r0 < r1), 1.0, 0.0).astype(jnp.float32)
    mt_block = jnp.where(same_b, 1.0, 0.0).astype(jnp.float32)
    fo = jnp.dot(p_f, mt_strict, preferred_element_type=jnp.float32,
                 precision=lax.Precision.HIGHEST)
    tf = jnp.dot(p_f, mt_block, preferred_element_type=jnp.float32,
                 precision=lax.Precision.HIGHEST)
    s_idx = lax.broadcasted_iota(jnp.int32, (1, R), 1) % S
    lo = s_idx.astype(jnp.float32) * MAX_LEN - fo
    pos_f = pos.astype(jnp.float32)
    dest_f = jnp.where(pos < p, fo + pos_f, tf + lo + pos_f - p_f)
    dest_ref[...] = dest_f.astype(jnp.int32)


_rnn_call = pl.pallas_call(
    _rnn_body,
    out_shape=[
        jax.ShapeDtypeStruct((MAX_LEN, R), jnp.int32),
        jax.ShapeDtypeStruct((MAX_LEN, R), jnp.float32),
        jax.ShapeDtypeStruct((MAX_LEN, R), jnp.float32),
        jax.ShapeDtypeStruct((MAX_LEN, R), jnp.int32),
    ],
)


_NC, _NS = 2, 16
_NW = _NC * _NS


def _sc_scatter_body(vals_f_hbm, toks_hbm, dest_hbm, out_f_hbm, out_i_hbm,
                     val_v, tok_v, orow_f, orow_i, idx_v):
    wid = lax.axis_index("s") * _NC + lax.axis_index("c")
    b = lax.rem(wid, B)
    pltpu.sync_copy(vals_f_hbm.at[wid], val_v)
    pltpu.sync_copy(dest_hbm.at[b], idx_v)

    def body_f(i, carry):
        sl = pl.ds(i * 16, 16)
        plsc.store_scatter(orow_f, [idx_v[sl]], val_v[sl])
        return carry

    lax.fori_loop(0, FLAT // 16, body_f, 0)
    pltpu.sync_copy(orow_f, out_f_hbm.at[wid])

    @pl.when(wid < B)
    def _():
        pltpu.sync_copy(toks_hbm.at[wid], tok_v)

        def body_i(i, carry):
            sl = pl.ds(i * 16, 16)
            plsc.store_scatter(orow_i, [idx_v[sl]], tok_v[sl])
            return carry

        lax.fori_loop(0, FLAT // 16, body_i, 0)
        pltpu.sync_copy(orow_i, out_i_hbm.at[wid])


@functools.cache
def _make_sc_scatter():
    return pl.kernel(
        _sc_scatter_body,
        mesh=plsc.VectorSubcoreMesh(
            core_axis_name="c", subcore_axis_name="s",
            num_cores=_NC, num_subcores=_NS),
        out_type=[
            jax.ShapeDtypeStruct((2 * B, FLAT), jnp.float32),
            jax.ShapeDtypeStruct((B, FLAT), jnp.int32),
        ],
        scratch_types=[
            pltpu.VMEM((FLAT,), jnp.float32),
            pltpu.VMEM((FLAT,), jnp.int32),
            pltpu.VMEM((FLAT,), jnp.float32),
            pltpu.VMEM((FLAT,), jnp.int32),
            pltpu.VMEM((FLAT,), jnp.int32),
        ],
        compiler_params=pltpu.CompilerParams(needs_layout_passes=False),
    )


@jax.jit
def kernel(input, W_in, W_h, W_out):
    xt = input.reshape(R, D_IN).T
    toks_t, lp_t, ent_t, dest_t = _rnn_call(xt, W_in.T, W_h.T, W_out.T)
    toks = toks_t.T.reshape(B, FLAT)
    dest = dest_t.T.reshape(B, FLAT)
    vals_f = jnp.concatenate(
        [lp_t.T.reshape(B, FLAT), ent_t.T.reshape(B, FLAT)], axis=0)
    out_f, out_i = _make_sc_scatter()(vals_f, toks, dest)
    return (out_i, out_f[:B], out_f[B:])


# speedup vs baseline: 38.7582x; 1.0347x over previous
import functools

import jax
import jax.numpy as jnp
from jax import lax
from jax.experimental import pallas as pl
from jax.experimental.pallas import tpu as pltpu
from jax.experimental.pallas import tpu_sc as plsc

B, S, D_IN, HID, VOCAB, MAX_LEN = 16, 8, 256, 512, 128, 512
R = B * S
FLAT = S * MAX_LEN


def _rnn_body(xt_ref, wint_ref, wht_ref, woutt_ref,
              tok_ref, lp_ref, ent_ref, dest_ref):
    h0 = jnp.tanh(jnp.dot(wint_ref[...], xt_ref[...],
                          preferred_element_type=jnp.float32))
    wht = wht_ref[...]
    woutt = woutt_ref[...]
    vocab_iota = lax.broadcasted_iota(jnp.int32, (VOCAB, R), 0)

    def step(t, h):
        h = jnp.tanh(jnp.dot(wht, h, preferred_element_type=jnp.float32))
        logits = jnp.dot(woutt, h, preferred_element_type=jnp.float32)
        m = jnp.max(logits, axis=0, keepdims=True)
        shifted = logits - m
        e = jnp.exp(shifted)
        ssum = jnp.sum(e, axis=0, keepdims=True)
        tok = jnp.min(jnp.where(logits == m, vocab_iota, VOCAB),
                      axis=0, keepdims=True)
        logs = jnp.log(ssum)
        lp = -logs
        ent = logs - jnp.sum(e * shifted, axis=0, keepdims=True) / ssum
        tok_ref[pl.ds(t, 1), :] = tok
        lp_ref[pl.ds(t, 1), :] = lp
        ent_ref[pl.ds(t, 1), :] = ent
        return h

    lax.fori_loop(0, MAX_LEN, step, h0, unroll=8)

    toks = tok_ref[...]
    pos = lax.broadcasted_iota(jnp.int32, (MAX_LEN, R), 0)
    fz = jnp.min(jnp.where(toks == 0, pos, MAX_LEN), axis=0, keepdims=True)
    length = jnp.minimum(fz + 1, MAX_LEN)
    p = length - 1
    p_f = p.astype(jnp.float32)

    r0 = lax.broadcasted_iota(jnp.int32, (R, R), 0)
    r1 = lax.broadcasted_iota(jnp.int32, (R, R), 1)
    same_b = (r0 // S) == (r1 // S)
    mt_strict = jnp.where(same_b & (r0 < r1), 1.0, 0.0).astype(jnp.float32)
    mt_block = jnp.where(same_b, 1.0, 0.0).astype(jnp.float32)
    fo = jnp.dot(p_f, mt_strict, preferred_element_type=jnp.float32,
                 precision=lax.Precision.HIGHEST)
    tf = jnp.dot(p_f, mt_block, preferred_element_type=jnp.float32,
                 precision=lax.Precision.HIGHEST)
    s_idx = lax.broadcasted_iota(jnp.int32, (1, R), 1) % S
    lo = s_idx.astype(jnp.float32) * MAX_LEN - fo
    pos_f = pos.astype(jnp.float32)
    dest_f = jnp.where(pos < p, fo + pos_f, tf + lo + pos_f - p_f)
    dest_ref[...] = dest_f.astype(jnp.int32)


_rnn_call = pl.pallas_call(
    _rnn_body,
    out_shape=[
        jax.ShapeDtypeStruct((MAX_LEN, R), jnp.int32),
        jax.ShapeDtypeStruct((MAX_LEN, R), jnp.float32),
        jax.ShapeDtypeStruct((MAX_LEN, R), jnp.float32),
        jax.ShapeDtypeStruct((MAX_LEN, R), jnp.int32),
    ],
)


_NC, _NS = 2, 16
_NW = _NC * _NS


def _sc_scatter_body(vals_f_hbm, toks_hbm, dest_hbm, out_f_hbm, out_i_hbm,
                     val_v, tok_v, orow_f, orow_i, idx_v):
    wid = lax.axis_index("s") * _NC + lax.axis_index("c")
    b = lax.rem(wid, B)
    pltpu.sync_copy(vals_f_hbm.at[wid], val_v)
    pltpu.sync_copy(dest_hbm.at[b], idx_v)

    def body_f(i, carry):
        sl = pl.ds(i * 16, 16)
        plsc.store_scatter(orow_f, [idx_v[sl]], val_v[sl])
        return carry

    lax.fori_loop(0, FLAT // 16, body_f, 0)
    pltpu.sync_copy(orow_f, out_f_hbm.at[wid])

    @pl.when(wid < B)
    def _():
        pltpu.sync_copy(toks_hbm.at[wid], tok_v)

        def body_i(i, carry):
            sl = pl.ds(i * 16, 16)
            plsc.store_scatter(orow_i, [idx_v[sl]], tok_v[sl])
            return carry

        lax.fori_loop(0, FLAT // 16, body_i, 0)
        pltpu.sync_copy(orow_i, out_i_hbm.at[wid])


@functools.cache
def _make_sc_scatter():
    return pl.kernel(
        _sc_scatter_body,
        mesh=plsc.VectorSubcoreMesh(
            core_axis_name="c", subcore_axis_name="s",
            num_cores=_NC, num_subcores=_NS),
        out_type=[
            jax.ShapeDtypeStruct((2 * B, FLAT), jnp.float32),
            jax.ShapeDtypeStruct((B, FLAT), jnp.int32),
        ],
        scratch_types=[
            pltpu.VMEM((FLAT,), jnp.float32),
            pltpu.VMEM((FLAT,), jnp.int32),
            pltpu.VMEM((FLAT,), jnp.float32),
            pltpu.VMEM((FLAT,), jnp.int32),
            pltpu.VMEM((FLAT,), jnp.int32),
        ],
        compiler_params=pltpu.CompilerParams(needs_layout_passes=False),
    )


@jax.jit
def kernel(input, W_in, W_h, W_out):
    xt = input.reshape(R, D_IN).T
    toks_t, lp_t, ent_t, dest_t = _rnn_call(xt, W_in.T, W_h.T, W_out.T)
    toks = toks_t.T.reshape(B, FLAT)
    dest = dest_t.T.reshape(B, FLAT)
    vals_f = jnp.concatenate(
        [lp_t.T.reshape(B, FLAT), ent_t.T.reshape(B, FLAT)], axis=0)
    out_f, out_i = _make_sc_scatter()(vals_f, toks, dest)
    return (out_i, out_f[:B], out_f[B:])
